# Initial kernel scaffold; baseline (speedup 1.0000x reference)
import jax
import jax.numpy as jnp
from jax.experimental import pallas as pl


def kernel(inputs, Wg, W, b):
    raise NotImplementedError("write your pallas kernel here")


# speedup vs baseline: 1.1393x; 1.1393x over previous
import functools

import jax
import jax.numpy as jnp
from jax.experimental import pallas as pl
from jax.experimental.pallas import tpu as pltpu


def _moe_dense_body(x_ref, wg_ref, w_ref, b_ref, out_ref, wts_ref, *, n_experts):
    e = pl.program_id(1)
    x = x_ref[...]

    @pl.when(e == 0)
    def _router():
        logits = jax.lax.dot_general(
            x, wg_ref[...], (((1,), (1,)), ((), ())),
            preferred_element_type=jnp.float32)
        ids = jax.lax.broadcasted_iota(jnp.int32, logits.shape, 1)
        m1 = jnp.max(logits, axis=1, keepdims=True)
        a1 = jnp.min(jnp.where(logits == m1, ids, n_experts), axis=1,
                     keepdims=True)
        l2 = jnp.where(ids == a1, -jnp.inf, logits)
        m2 = jnp.max(l2, axis=1, keepdims=True)
        a2 = jnp.min(jnp.where(l2 == m2, ids, n_experts), axis=1, keepdims=True)
        z = jnp.exp(m2 - m1)
        w1 = 1.0 / (1.0 + z)
        w2 = z / (1.0 + z)
        wts_ref[...] = (jnp.where(ids == a1, w1, 0.0)
                        + jnp.where(ids == a2, w2, 0.0))
        out_ref[...] = jnp.zeros_like(out_ref)

    xb = x.astype(jnp.bfloat16)
    y = jax.lax.dot_general(
        xb, w_ref[0], (((1,), (1,)), ((), ())),
        preferred_element_type=jnp.float32)
    ids = jax.lax.broadcasted_iota(jnp.int32, wts_ref.shape, 1)
    we = jnp.sum(wts_ref[...] * (ids == e).astype(jnp.float32), axis=1,
                 keepdims=True)
    out_ref[...] += (y + b_ref[0]) * we


def kernel(inputs, Wg, W, b):
    B, S, D_IN = inputs.shape
    E, D_OUT, _ = W.shape
    T = B * S
    BT = 512

    x = inputs.reshape(T, D_IN)
    Wbf = W.astype(jnp.bfloat16)
    b3 = b.reshape(E, 1, D_OUT)

    out = pl.pallas_call(
        functools.partial(_moe_dense_body, n_experts=E),
        grid=(T // BT, E),
        in_specs=[
            pl.BlockSpec((BT, D_IN), lambda t, e: (t, 0)),
            pl.BlockSpec((E, D_IN), lambda t, e: (0, 0)),
            pl.BlockSpec((1, D_OUT, D_IN), lambda t, e: (e, 0, 0)),
            pl.BlockSpec((1, 1, D_OUT), lambda t, e: (e, 0, 0)),
        ],
        out_specs=pl.BlockSpec((BT, D_OUT), lambda t, e: (t, 0)),
        out_shape=jax.ShapeDtypeStruct((T, D_OUT), jnp.float32),
        scratch_shapes=[pltpu.VMEM((BT, E), jnp.float32)],
        compiler_params=pltpu.CompilerParams(
            dimension_semantics=("arbitrary", "arbitrary")),
    )(x, Wg, Wbf, b3)
    return out.reshape(B, S, D_OUT)


# speedup vs baseline: 1.1814x; 1.0370x over previous
import functools

import jax
import jax.numpy as jnp
from jax.experimental import pallas as pl
from jax.experimental.pallas import tpu as pltpu


def _moe_dense_body(x_ref, wg_ref, w_ref, b_ref, out_ref, wts_ref, *, n_experts):
    e = pl.program_id(1)
    x = x_ref[...]

    @pl.when(e == 0)
    def _router():
        logits = jax.lax.dot_general(
            x, wg_ref[...], (((1,), (1,)), ((), ())),
            preferred_element_type=jnp.float32)
        ids = jax.lax.broadcasted_iota(jnp.int32, logits.shape, 1)
        m1 = jnp.max(logits, axis=1, keepdims=True)
        a1 = jnp.min(jnp.where(logits == m1, ids, n_experts), axis=1,
                     keepdims=True)
        l2 = jnp.where(ids == a1, -jnp.inf, logits)
        m2 = jnp.max(l2, axis=1, keepdims=True)
        a2 = jnp.min(jnp.where(l2 == m2, ids, n_experts), axis=1, keepdims=True)
        z = jnp.exp(m2 - m1)
        w1 = 1.0 / (1.0 + z)
        w2 = z / (1.0 + z)
        wts_ref[...] = (jnp.where(ids == a1, w1, 0.0)
                        + jnp.where(ids == a2, w2, 0.0))
        out_ref[...] = jnp.zeros_like(out_ref)

    xb = x.astype(jnp.bfloat16)
    y = jax.lax.dot_general(
        xb, w_ref[0], (((1,), (1,)), ((), ())),
        preferred_element_type=jnp.float32)
    ids = jax.lax.broadcasted_iota(jnp.int32, wts_ref.shape, 1)
    we = jnp.sum(wts_ref[...] * (ids == e).astype(jnp.float32), axis=1,
                 keepdims=True)
    out_ref[...] += (y + b_ref[0]) * we


def kernel(inputs, Wg, W, b):
    B, S, D_IN = inputs.shape
    E, D_OUT, _ = W.shape
    T = B * S
    BT = 1024

    x = inputs.reshape(T, D_IN)
    Wbf = W.astype(jnp.bfloat16)
    b3 = b.reshape(E, 1, D_OUT)

    out = pl.pallas_call(
        functools.partial(_moe_dense_body, n_experts=E),
        grid=(T // BT, E),
        in_specs=[
            pl.BlockSpec((BT, D_IN), lambda t, e: (t, 0)),
            pl.BlockSpec((E, D_IN), lambda t, e: (0, 0)),
            pl.BlockSpec((1, D_OUT, D_IN), lambda t, e: (e, 0, 0)),
            pl.BlockSpec((1, 1, D_OUT), lambda t, e: (e, 0, 0)),
        ],
        out_specs=pl.BlockSpec((BT, D_OUT), lambda t, e: (t, 0)),
        out_shape=jax.ShapeDtypeStruct((T, D_OUT), jnp.float32),
        scratch_shapes=[pltpu.VMEM((BT, E), jnp.float32)],
        compiler_params=pltpu.CompilerParams(
            dimension_semantics=("arbitrary", "arbitrary"),
            vmem_limit_bytes=100 * 1024 * 1024),
    )(x, Wg, Wbf, b3)
    return out.reshape(B, S, D_OUT)
